# baseline (speedup 1.0000x reference)
import functools

import jax
import jax.numpy as jnp
from jax import lax
from jax.experimental import pallas as pl
from jax.experimental.pallas import tpu as pltpu
from jax.experimental.pallas import tpu_sc as plsc


_BLK = 2000


def _proj_body(nblk_node, xg_ref, ea_ref, wn_ref, w1_ref, w2_ref,
               bn_ref, be_ref, t_ref, base_ref):
    i = pl.program_id(0)

    @pl.when(i < nblk_node)
    def _():
        t_ref[...] = (
            jnp.dot(xg_ref[...], wn_ref[...],
                    preferred_element_type=jnp.float32) + bn_ref[...])

    @pl.when(i >= nblk_node)
    def _():
        ea = ea_ref[...]
        t_ref[...] = jnp.dot(ea, w1_ref[...],
                             preferred_element_type=jnp.float32)
        z = (jnp.dot(ea, w2_ref[...], preferred_element_type=jnp.float32)
             + be_ref[...])
        base_ref[...] = jnp.where(z > 0, z, jnp.exp(z) - 1.0)


def _fused_proj(xg_pad, edge_attr, w_node, w1, w2, b_edge, b_e):
    n, kn = xg_pad.shape
    e, ke = edge_attr.shape
    nbn = n // _BLK
    nbe = e // _BLK
    t_rows, base = pl.pallas_call(
        functools.partial(_proj_body, nbn),
        grid=(nbn + nbe,),
        in_specs=[
            pl.BlockSpec((_BLK, kn), lambda i: (jnp.minimum(i, nbn - 1), 0)),
            pl.BlockSpec((_BLK, ke),
                         lambda i: (jnp.maximum(i - nbn, 0), 0)),
            pl.BlockSpec((kn, 128), lambda i: (0, 0)),
            pl.BlockSpec((ke, 128), lambda i: (0, 0)),
            pl.BlockSpec((ke, 128), lambda i: (0, 0)),
            pl.BlockSpec((1, 128), lambda i: (0, 0)),
            pl.BlockSpec((1, 128), lambda i: (0, 0)),
        ],
        out_specs=[
            pl.BlockSpec((_BLK, 128), lambda i: (i, 0)),
            pl.BlockSpec((_BLK, 128),
                         lambda i: (jnp.maximum(i - nbn, 0), 0)),
        ],
        out_shape=[
            jax.ShapeDtypeStruct((n + e, 128), jnp.float32),
            jax.ShapeDtypeStruct((e, 128), jnp.float32),
        ],
    )(xg_pad, edge_attr, w_node, w1, w2,
      b_edge.reshape(1, 128), b_e.reshape(1, 128))
    return t_rows, base


_C = 10000
_STEPS = 40
_BE = 2000
_NB = 25
_NV = _BE // 16
_KB = 64
_LR = 64
_SV = 125


def _sc_stage(t_rows, base, e0, e1, a0, nn):
    n_edges = base.shape[0]
    tile_e = n_edges // 16
    rows_pt = _C // 16

    def body(t_hbm, base_hbm, e0_hbm, e1_hbm, a0_hbm, out_hbm,
             acc, e0_buf, pos2d, seg2d, e1s, a0s, cmb, rbuf,
             semi, semr, sems, seme):
        c = lax.axis_index("c")
        s = lax.axis_index("s")
        tile_base = s * tile_e

        def fire_idx(d):
            p = lax.bitwise_and(d, 1)
            pltpu.async_copy(e1_hbm.at[pos2d.at[pl.ds(d * _KB, _KB)]], e1s.at[p], semi.at[p])
            pltpu.async_copy(a0_hbm.at[pos2d.at[pl.ds(d * _KB, _KB)]], a0s.at[p], semi.at[p])

        def wait_idx(d):
            p = lax.bitwise_and(d, 1)
            pltpu.make_async_copy(
                e1_hbm.at[pos2d.at[pl.ds(d * _KB, _KB)]], e1s.at[p], semi.at[p]).wait()
            pltpu.make_async_copy(
                a0_hbm.at[pos2d.at[pl.ds(d * _KB, _KB)]], a0s.at[p], semi.at[p]).wait()

        def build_fire_rg(d):
            p = lax.bitwise_and(d, 1)
            for j in range(_KB // 16):
                cmb[p, pl.ds(j * 16, 16)] = e1s[p, pl.ds(j * 16, 16)] + nn
                cmb[p, pl.ds(_KB + j * 16, 16)] = a0s[p, pl.ds(j * 16, 16)]
            pltpu.async_copy(t_hbm.at[cmb.at[p]], rbuf.at[p], semr.at[p])

        def finish(d):
            p = lax.bitwise_and(d, 1)
            pltpu.make_async_copy(
                t_hbm.at[cmb.at[p]], rbuf.at[p], semr.at[p]).wait()

            def elu_body(i, _):
                r = i // 8
                o = (i % 8) * 16
                vv = rbuf[p, r, pl.ds(o, 16)] + rbuf[p, _KB + r, pl.ds(o, 16)]
                vv = jnp.where(vv > 0, vv, jnp.exp(vv) - 1.0)
                rbuf[p, r, pl.ds(o, 16)] = vv
                return 0

            lax.fori_loop(0, _KB * 8, elu_body, 0, unroll=8)
            pltpu.async_copy(rbuf.at[p, pl.ds(0, _KB)],
                             acc.at[seg2d.at[pl.ds(d * _KB, _KB)]], sems.at[p], add=True)

        def wait_s(d):
            p = lax.bitwise_and(d, 1)
            pltpu.make_async_copy(rbuf.at[p, pl.ds(0, _KB)],
                                  acc.at[seg2d.at[pl.ds(d * _KB, _KB)]], sems.at[p]).wait()

        def drain_rows(n):
            fire_idx(0)

            def pipe(d, _):
                @pl.when(d > 1)
                def _():
                    wait_s(d - 2)
                wait_idx(d)
                build_fire_rg(d)

                @pl.when(d + 1 < n)
                def _():
                    fire_idx(d + 1)

                @pl.when(d > 0)
                def _():
                    finish(d - 1)
                return 0

            lax.fori_loop(0, n, pipe, 0)

            @pl.when(n > 1)
            def _():
                wait_s(n - 2)
            finish(n - 1)
            wait_s(n - 1)

        def chunk_body(step, _):
            lo = (c * _STEPS + step) * _C
            pltpu.sync_copy(
                base_hbm.at[pl.ds(lo + s * rows_pt, rows_pt)],
                acc.at[pl.ds(s * rows_pt, rows_pt)])
            plsc.subcore_barrier()

            def fire_e0(b):
                p = lax.bitwise_and(b, 1)
                pltpu.async_copy(
                    e0_hbm.at[pl.ds(tile_base + b * _BE, _BE)],
                    e0_buf.at[p], seme.at[p])

            fire_e0(0)

            def block_body(b, cnt_in):
                eb = tile_base + b * _BE
                p0 = lax.bitwise_and(b, 1)
                pltpu.make_async_copy(
                    e0_hbm.at[pl.ds(eb, _BE)], e0_buf.at[p0],
                    seme.at[p0]).wait()

                @pl.when(b + 1 < _NB)
                def _():
                    fire_e0(b + 1)

                def sub_body(b2, cnt_sub):
                    def vreg_body(i, carry):
                        cnt, posv = carry
                        v = b2 * _SV + i
                        ev = e0_buf[p0, pl.ds(v * 16, 16)]
                        rel = ev - lo
                        m = (rel >= 0) & (rel < _C)
                        inc = jnp.sum(m.astype(jnp.int32))
                        plsc.store_compressed(
                            pos2d.at[pl.ds(cnt, 16)], posv, mask=m)
                        plsc.store_compressed(
                            seg2d.at[pl.ds(cnt, 16)], rel, mask=m)
                        return cnt + inc, posv + 16

                    posv0 = eb + b2 * _SV * 16 + lax.iota(jnp.int32, 16)
                    cnt2, _p = lax.fori_loop(
                        0, _SV, vreg_body, (cnt_sub, posv0), unroll=8)

                    @pl.when(cnt2 >= _LR * _KB - _SV * 16)
                    def _():
                        nfull = lax.shift_right_logical(cnt2, 6)
                        tb = nfull * _KB
                        drain_rows(nfull)
                        for j in range(4):
                            pos2d[pl.ds(j * 16, 16)] = (
                                pos2d[pl.ds(tb + j * 16, 16)])
                            seg2d[pl.ds(j * 16, 16)] = (
                                seg2d[pl.ds(tb + j * 16, 16)])

                    return jnp.where(cnt2 >= _LR * _KB - _SV * 16,
                                     lax.bitwise_and(cnt2, 63), cnt2)

                return lax.fori_loop(0, _NV // _SV, sub_body, cnt_in)

            cnt = lax.fori_loop(0, _NB, block_body, jnp.int32(0))

            n_dr = (cnt + _KB - 1) // _KB
            zero16 = jnp.zeros((16,), jnp.int32)
            trash16 = jnp.full((16,), _C, jnp.int32)
            for j in range(4):
                pos2d[pl.ds(cnt + j * 16, 16)] = zero16
                seg2d[pl.ds(cnt + j * 16, 16)] = trash16

            @pl.when(n_dr > 0)
            def _():
                drain_rows(n_dr)

            plsc.subcore_barrier()
            pltpu.sync_copy(
                acc.at[pl.ds(s * rows_pt, rows_pt)],
                out_hbm.at[pl.ds(lo + s * rows_pt, rows_pt)])
            plsc.subcore_barrier()
            return 0

        lax.fori_loop(0, _STEPS, chunk_body, 0)

    run = pl.kernel(
        body,
        out_type=jax.ShapeDtypeStruct((n_edges, 128), jnp.float32),
        mesh=plsc.VectorSubcoreMesh(core_axis_name="c", subcore_axis_name="s"),
        compiler_params=pltpu.CompilerParams(
            needs_layout_passes=False, use_tc_tiling_on_sc=False),
        scratch_types=[
            pltpu.VMEM_SHARED((_C + 8, 128), jnp.float32),
            pltpu.VMEM((2, _BE), jnp.int32),
            pltpu.VMEM((_LR * _KB + _KB,), jnp.int32),
            pltpu.VMEM((_LR * _KB + _KB,), jnp.int32),
            pltpu.VMEM((2, _KB), jnp.int32),
            pltpu.VMEM((2, _KB), jnp.int32),
            pltpu.VMEM((2, 2 * _KB), jnp.int32),
            pltpu.VMEM((2, 2 * _KB, 128), jnp.float32),
            pltpu.SemaphoreType.DMA((2,)),
            pltpu.SemaphoreType.DMA((2,)),
            pltpu.SemaphoreType.DMA((2,)),
            pltpu.SemaphoreType.DMA((2,)),
        ],
    )
    return run(t_rows, base, e0, e1, a0)


def kernel(x, edge_attr, atom_index, e_idx, global_state, W_edge, b_edge, W_e, b_e):
    xg = jnp.concatenate([x, global_state], axis=1)
    k_node = xg.shape[1]
    k_pad = 64
    xg_pad = jnp.pad(xg, ((0, 0), (0, k_pad - k_node)))
    w_node = jnp.pad(W_edge[edge_attr.shape[1]:], ((0, k_pad - k_node), (0, 0)))

    t_rows, base = _fused_proj(xg_pad, edge_attr, w_node,
                               W_edge[:edge_attr.shape[1]], W_e, b_edge, b_e)
    return _sc_stage(t_rows, base, e_idx[0], e_idx[1], atom_index[0],
                     x.shape[0])


# speedup vs baseline: 1.0566x; 1.0566x over previous
import functools

import jax
import jax.numpy as jnp
from jax import lax
from jax.experimental import pallas as pl
from jax.experimental.pallas import tpu as pltpu
from jax.experimental.pallas import tpu_sc as plsc


_BLK = 2000


def _proj_body(nblk_node, xg_ref, ea_ref, wn_ref, w1_ref, w2_ref,
               bn_ref, be_ref, t_ref, base_ref):
    i = pl.program_id(0)

    @pl.when(i < nblk_node)
    def _():
        t_ref[...] = (
            jnp.dot(xg_ref[...], wn_ref[...],
                    preferred_element_type=jnp.float32) + bn_ref[...])

    @pl.when(i >= nblk_node)
    def _():
        ea = ea_ref[...]
        t_ref[...] = jnp.dot(ea, w1_ref[...],
                             preferred_element_type=jnp.float32)
        z = (jnp.dot(ea, w2_ref[...], preferred_element_type=jnp.float32)
             + be_ref[...])
        base_ref[...] = jnp.where(z > 0, z, jnp.exp(z) - 1.0)


def _fused_proj(xg_pad, edge_attr, w_node, w1, w2, b_edge, b_e):
    n, kn = xg_pad.shape
    e, ke = edge_attr.shape
    nbn = n // _BLK
    nbe = e // _BLK
    t_rows, base = pl.pallas_call(
        functools.partial(_proj_body, nbn),
        grid=(nbn + nbe,),
        in_specs=[
            pl.BlockSpec((_BLK, kn), lambda i: (jnp.minimum(i, nbn - 1), 0)),
            pl.BlockSpec((_BLK, ke),
                         lambda i: (jnp.maximum(i - nbn, 0), 0)),
            pl.BlockSpec((kn, 128), lambda i: (0, 0)),
            pl.BlockSpec((ke, 128), lambda i: (0, 0)),
            pl.BlockSpec((ke, 128), lambda i: (0, 0)),
            pl.BlockSpec((1, 128), lambda i: (0, 0)),
            pl.BlockSpec((1, 128), lambda i: (0, 0)),
        ],
        out_specs=[
            pl.BlockSpec((_BLK, 128), lambda i: (i, 0)),
            pl.BlockSpec((_BLK, 128),
                         lambda i: (jnp.maximum(i - nbn, 0), 0)),
        ],
        out_shape=[
            jax.ShapeDtypeStruct((n + e, 128), jnp.float32),
            jax.ShapeDtypeStruct((e, 128), jnp.float32),
        ],
    )(xg_pad, edge_attr, w_node, w1, w2,
      b_edge.reshape(1, 128), b_e.reshape(1, 128))
    return t_rows, base


_C = 10000
_STEPS = 40
_BE = 2000
_NB = 25
_NV = _BE // 16
_KB = 64
_LR = 64
_SV = 125


def _sc_stage(t_rows, base, e0, e1, a0, nn):
    n_edges = base.shape[0]
    tile_e = n_edges // 16
    rows_pt = _C // 16

    def body(t_hbm, base_hbm, e0_hbm, e1_hbm, a0_hbm, out_hbm,
             acc, e0_buf, pos2d, seg2d, e1s, a0s, cmb, rbuf,
             semi, semr, sems, seme, semf):
        c = lax.axis_index("c")
        s = lax.axis_index("s")
        tile_base = s * tile_e

        def fire_idx(d):
            p = lax.bitwise_and(d, 1)
            pltpu.async_copy(e1_hbm.at[pos2d.at[pl.ds(d * _KB, _KB)]], e1s.at[p], semi.at[p])
            pltpu.async_copy(a0_hbm.at[pos2d.at[pl.ds(d * _KB, _KB)]], a0s.at[p], semi.at[p])

        def wait_idx(d):
            p = lax.bitwise_and(d, 1)
            pltpu.make_async_copy(
                e1_hbm.at[pos2d.at[pl.ds(d * _KB, _KB)]], e1s.at[p], semi.at[p]).wait()
            pltpu.make_async_copy(
                a0_hbm.at[pos2d.at[pl.ds(d * _KB, _KB)]], a0s.at[p], semi.at[p]).wait()

        def build_fire_rg(d):
            p = lax.bitwise_and(d, 1)
            for j in range(_KB // 16):
                cmb[p, pl.ds(j * 16, 16)] = e1s[p, pl.ds(j * 16, 16)] + nn
                cmb[p, pl.ds(_KB + j * 16, 16)] = a0s[p, pl.ds(j * 16, 16)]
            pltpu.async_copy(t_hbm.at[cmb.at[p]], rbuf.at[p], semr.at[p])

        def finish(d):
            p = lax.bitwise_and(d, 1)
            pltpu.make_async_copy(
                t_hbm.at[cmb.at[p]], rbuf.at[p], semr.at[p]).wait()

            def elu_body(i, _):
                r = i // 8
                o = (i % 8) * 16
                vv = rbuf[p, r, pl.ds(o, 16)] + rbuf[p, _KB + r, pl.ds(o, 16)]
                vv = jnp.where(vv > 0, vv, jnp.exp(vv) - 1.0)
                rbuf[p, r, pl.ds(o, 16)] = vv
                return 0

            lax.fori_loop(0, _KB * 8, elu_body, 0, unroll=8)
            pltpu.async_copy(rbuf.at[p, pl.ds(0, _KB)],
                             acc.at[seg2d.at[pl.ds(d * _KB, _KB)]], sems.at[p], add=True)

        def wait_s(d):
            p = lax.bitwise_and(d, 1)
            pltpu.make_async_copy(rbuf.at[p, pl.ds(0, _KB)],
                                  acc.at[seg2d.at[pl.ds(d * _KB, _KB)]], sems.at[p]).wait()

        def drain_rows(n):
            fire_idx(0)

            def pipe(d, _):
                @pl.when(d > 1)
                def _():
                    wait_s(d - 2)
                wait_idx(d)
                build_fire_rg(d)

                @pl.when(d + 1 < n)
                def _():
                    fire_idx(d + 1)

                @pl.when(d > 0)
                def _():
                    finish(d - 1)
                return 0

            lax.fori_loop(0, n, pipe, 0)

            @pl.when(n > 1)
            def _():
                wait_s(n - 2)
            finish(n - 1)
            wait_s(n - 1)

        def chunk_body(step, _):
            lo = (c * _STEPS + step) * _C

            @pl.when(step > 0)
            def _():
                lop = lo - _C
                pltpu.make_async_copy(
                    acc.at[pl.ds(s * rows_pt, rows_pt)],
                    out_hbm.at[pl.ds(lop + s * rows_pt, rows_pt)],
                    semf.at[0]).wait()
            pltpu.async_copy(
                base_hbm.at[pl.ds(lo + s * rows_pt, rows_pt)],
                acc.at[pl.ds(s * rows_pt, rows_pt)], semf.at[1])

            def fire_e0(b):
                p = lax.bitwise_and(b, 1)
                pltpu.async_copy(
                    e0_hbm.at[pl.ds(tile_base + b * _BE, _BE)],
                    e0_buf.at[p], seme.at[p])

            fire_e0(0)

            def block_body(b, cnt_in):
                eb = tile_base + b * _BE
                p0 = lax.bitwise_and(b, 1)
                pltpu.make_async_copy(
                    e0_hbm.at[pl.ds(eb, _BE)], e0_buf.at[p0],
                    seme.at[p0]).wait()

                @pl.when(b + 1 < _NB)
                def _():
                    fire_e0(b + 1)

                def sub_body(b2, cnt_sub):
                    def vreg_body(i, carry):
                        cnt, posv = carry
                        v = b2 * _SV + i
                        ev = e0_buf[p0, pl.ds(v * 16, 16)]
                        rel = ev - lo
                        m = (rel >= 0) & (rel < _C)
                        inc = jnp.sum(m.astype(jnp.int32))
                        plsc.store_compressed(
                            pos2d.at[pl.ds(cnt, 16)], posv, mask=m)
                        plsc.store_compressed(
                            seg2d.at[pl.ds(cnt, 16)], rel, mask=m)
                        return cnt + inc, posv + 16

                    posv0 = eb + b2 * _SV * 16 + lax.iota(jnp.int32, 16)
                    cnt2, _p = lax.fori_loop(
                        0, _SV, vreg_body, (cnt_sub, posv0), unroll=4)

                    @pl.when(cnt2 >= _LR * _KB - _SV * 16)
                    def _():
                        nfull = lax.shift_right_logical(cnt2, 6)
                        tb = nfull * _KB
                        drain_rows(nfull)
                        for j in range(4):
                            pos2d[pl.ds(j * 16, 16)] = (
                                pos2d[pl.ds(tb + j * 16, 16)])
                            seg2d[pl.ds(j * 16, 16)] = (
                                seg2d[pl.ds(tb + j * 16, 16)])

                    return jnp.where(cnt2 >= _LR * _KB - _SV * 16,
                                     lax.bitwise_and(cnt2, 63), cnt2)

                return lax.fori_loop(0, _NV // _SV, sub_body, cnt_in)

            cnt = block_body(0, jnp.int32(0))
            pltpu.make_async_copy(
                base_hbm.at[pl.ds(lo + s * rows_pt, rows_pt)],
                acc.at[pl.ds(s * rows_pt, rows_pt)], semf.at[1]).wait()
            plsc.subcore_barrier()
            cnt = lax.fori_loop(1, _NB, block_body, cnt)

            n_dr = (cnt + _KB - 1) // _KB
            zero16 = jnp.zeros((16,), jnp.int32)
            trash16 = jnp.full((16,), _C, jnp.int32)
            for j in range(4):
                pos2d[pl.ds(cnt + j * 16, 16)] = zero16
                seg2d[pl.ds(cnt + j * 16, 16)] = trash16

            @pl.when(n_dr > 0)
            def _():
                drain_rows(n_dr)

            plsc.subcore_barrier()
            pltpu.async_copy(
                acc.at[pl.ds(s * rows_pt, rows_pt)],
                out_hbm.at[pl.ds(lo + s * rows_pt, rows_pt)], semf.at[0])
            return 0

        lax.fori_loop(0, _STEPS, chunk_body, 0)
        lo_last = (c * _STEPS + _STEPS - 1) * _C
        pltpu.make_async_copy(
            acc.at[pl.ds(s * rows_pt, rows_pt)],
            out_hbm.at[pl.ds(lo_last + s * rows_pt, rows_pt)],
            semf.at[0]).wait()
        plsc.subcore_barrier()

    run = pl.kernel(
        body,
        out_type=jax.ShapeDtypeStruct((n_edges, 128), jnp.float32),
        mesh=plsc.VectorSubcoreMesh(core_axis_name="c", subcore_axis_name="s"),
        compiler_params=pltpu.CompilerParams(
            needs_layout_passes=False, use_tc_tiling_on_sc=False),
        scratch_types=[
            pltpu.VMEM_SHARED((_C + 8, 128), jnp.float32),
            pltpu.VMEM((2, _BE), jnp.int32),
            pltpu.VMEM((_LR * _KB + _KB,), jnp.int32),
            pltpu.VMEM((_LR * _KB + _KB,), jnp.int32),
            pltpu.VMEM((2, _KB), jnp.int32),
            pltpu.VMEM((2, _KB), jnp.int32),
            pltpu.VMEM((2, 2 * _KB), jnp.int32),
            pltpu.VMEM((2, 2 * _KB, 128), jnp.float32),
            pltpu.SemaphoreType.DMA((2,)),
            pltpu.SemaphoreType.DMA((2,)),
            pltpu.SemaphoreType.DMA((2,)),
            pltpu.SemaphoreType.DMA((2,)),
            pltpu.SemaphoreType.DMA((2,)),
        ],
    )
    return run(t_rows, base, e0, e1, a0)


def kernel(x, edge_attr, atom_index, e_idx, global_state, W_edge, b_edge, W_e, b_e):
    xg = jnp.concatenate([x, global_state], axis=1)
    k_node = xg.shape[1]
    k_pad = 64
    xg_pad = jnp.pad(xg, ((0, 0), (0, k_pad - k_node)))
    w_node = jnp.pad(W_edge[edge_attr.shape[1]:], ((0, k_pad - k_node), (0, 0)))

    t_rows, base = _fused_proj(xg_pad, edge_attr, w_node,
                               W_edge[:edge_attr.shape[1]], W_e, b_edge, b_e)
    return _sc_stage(t_rows, base, e_idx[0], e_idx[1], atom_index[0],
                     x.shape[0])
